# baseline (speedup 1.0000x reference)
import jax
import jax.numpy as jnp
from jax.experimental import pallas as pl
from jax.experimental.pallas import tpu as pltpu

_N_CHUNKS = 8


def _dma_body(w_ref, o_ref, *sems):
    rows = w_ref.shape[0]
    chunk = rows // _N_CHUNKS
    copies = [
        pltpu.make_async_copy(
            w_ref.at[pl.ds(i * chunk, chunk), :],
            o_ref.at[pl.ds(i * chunk, chunk), :],
            sems[i],
        )
        for i in range(_N_CHUNKS)
    ]
    for c in copies:
        c.start()
    for c in copies:
        c.wait()


def kernel(x, embed_weight):
    seq_len = x.shape[1]
    n_model = embed_weight.shape[1]
    table = embed_weight[:seq_len]
    return pl.pallas_call(
        _dma_body,
        in_specs=[pl.BlockSpec(memory_space=pl.ANY)],
        out_specs=pl.BlockSpec(memory_space=pl.ANY),
        out_shape=jax.ShapeDtypeStruct((seq_len, n_model), embed_weight.dtype),
        scratch_shapes=[pltpu.SemaphoreType.DMA] * _N_CHUNKS,
    )(table)


# speedup vs baseline: 46.7197x; 46.7197x over previous
import jax
import jax.numpy as jnp
from jax.experimental import pallas as pl
from jax.experimental.pallas import tpu as pltpu

_N_CHUNKS = 16


def _dma_body(w_ref, o_ref, scratch, in_sems, out_sems):
    rows = w_ref.shape[0]
    chunk = rows // _N_CHUNKS
    ins = [
        pltpu.make_async_copy(
            w_ref.at[pl.ds(i * chunk, chunk), :],
            scratch.at[pl.ds(i * chunk, chunk), :],
            in_sems.at[i],
        )
        for i in range(_N_CHUNKS)
    ]
    outs = [
        pltpu.make_async_copy(
            scratch.at[pl.ds(i * chunk, chunk), :],
            o_ref.at[pl.ds(i * chunk, chunk), :],
            out_sems.at[i],
        )
        for i in range(_N_CHUNKS)
    ]
    for c in ins:
        c.start()
    for i in range(_N_CHUNKS):
        ins[i].wait()
        outs[i].start()
    for c in outs:
        c.wait()


def kernel(x, embed_weight):
    seq_len = x.shape[1]
    n_model = embed_weight.shape[1]
    table = embed_weight[:seq_len]
    return pl.pallas_call(
        _dma_body,
        in_specs=[pl.BlockSpec(memory_space=pl.ANY)],
        out_specs=pl.BlockSpec(memory_space=pl.ANY),
        out_shape=jax.ShapeDtypeStruct((seq_len, n_model), embed_weight.dtype),
        scratch_shapes=[
            pltpu.VMEM((seq_len, n_model), embed_weight.dtype),
            pltpu.SemaphoreType.DMA((_N_CHUNKS,)),
            pltpu.SemaphoreType.DMA((_N_CHUNKS,)),
        ],
    )(table)
